# Initial kernel scaffold; baseline (speedup 1.0000x reference)
import jax
import jax.numpy as jnp
from jax.experimental import pallas as pl


def kernel(query, gaussian_means, lidar2img, feat0, feat1, feat2, feat3, dmap0, dmap1, dmap2, dmap3, W_off3d, b_off3d, W_uvd, b_uvd, W_attn, b_attn, W_out, b_out, H_orig, W_orig):
    raise NotImplementedError("write your pallas kernel here")


# speedup vs baseline: 1.1929x; 1.1929x over previous
import functools

import jax
import jax.numpy as jnp
from jax.experimental import pallas as pl

_B, _Nc, _N, _C = 1, 6, 1024, 256
_M, _HEAD, _L, _NR1, _NR2, _DBINS = 8, 32, 4, 2, 4, 32
_DMIN, _DMAX = 1.0, 60.0
_LEVEL_HW = [(32, 88), (16, 44), (8, 22), (4, 11)]


def _out_proj_body(x_ref, w_ref, b_ref, q_ref, o_ref):
    o_ref[...] = (
        jnp.dot(x_ref[...], w_ref[...], preferred_element_type=jnp.float32)
        + b_ref[...][None, :] + q_ref[...]
    )


def _out_projection(x, w, b, q):
    return pl.pallas_call(
        _out_proj_body,
        out_shape=jax.ShapeDtypeStruct((_N, _C), jnp.float32),
    )(x, w, b, q)


def kernel(query, gaussian_means, lidar2img, feat0, feat1, feat2, feat3,
           dmap0, dmap1, dmap2, dmap3, W_off3d, b_off3d, W_uvd, b_uvd,
           W_attn, b_attn, W_out, b_out, H_orig, W_orig):
    feats = [feat0, feat1, feat2, feat3]
    dmaps = [dmap0, dmap1, dmap2, dmap3]
    q = query[0]
    gm = gaussian_means[0]
    l2i = lidar2img[0]

    off3d = (q @ W_off3d + b_off3d).reshape(_N, _NR1, 3)
    ref3d = gm[:, None, :] + off3d
    homo = jnp.concatenate(
        [ref3d, jnp.ones((_N, _NR1, 1), q.dtype)], -1).reshape(_N * _NR1, 4)
    proj = jnp.einsum('cij,nj->cni', l2i, homo)
    depth = jnp.maximum(proj[..., 2], 1e-5)
    u = proj[..., 0] / depth / W_orig
    v = proj[..., 1] / depth / H_orig
    dn = jnp.clip((depth - _DMIN) / (_DMAX - _DMIN), 0.0, 1.0)
    valid = ((u >= 0) & (u <= 1) & (v >= 0) & (v <= 1) & (depth > _DMIN))
    u = u.reshape(_Nc, _N, _NR1)
    v = v.reshape(_Nc, _N, _NR1)
    dn = dn.reshape(_Nc, _N, _NR1)
    valid = valid.reshape(_Nc, _N, _NR1)

    offs = (q @ W_uvd + b_uvd).reshape(_N, _M, _L, _NR1, _NR2, 3)
    attn = (q @ W_attn + b_attn).reshape(_N, _M, _L * _NR1 * _NR2)
    attn = jax.nn.softmax(attn, axis=-1).reshape(_N, _M, _L, _NR1, _NR2)

    su = u[:, :, None, None, :, None] + offs[None, :, :, :, :, :, 0]
    sv = v[:, :, None, None, :, None] + offs[None, :, :, :, :, :, 1]
    sd = dn[:, :, None, None, :, None] + offs[None, :, :, :, :, :, 2]
    w_all = attn[None] * valid[:, :, None, None, :, None].astype(q.dtype)

    m_idx = jnp.arange(_M, dtype=jnp.int32)[None, None, :, None, None]

    acc = jnp.zeros((_Nc, _N, _M, _HEAD), jnp.float32)
    for l, (H, W) in enumerate(_LEVEL_HW):
        HW = H * W
        featfl = feats[l].transpose(0, 2, 3, 1).reshape(_Nc, HW * _M, _HEAD)
        dmapfl = dmaps[l].transpose(0, 2, 3, 1).reshape(_Nc, HW * _DBINS)
        x = su[:, :, :, l] * W - 0.5
        y = sv[:, :, :, l] * H - 0.5
        d = jnp.clip(sd[:, :, :, l], 0.0, 1.0)
        wl = w_all[:, :, :, l]
        P = _N * _M * _NR1 * _NR2

        x0 = jnp.floor(x)
        y0 = jnp.floor(y)
        fx = x - x0
        fy = y - y0
        x0 = x0.astype(jnp.int32)
        y0 = y0.astype(jnp.int32)
        dc = d * (_DBINS - 1)
        dlo = jnp.clip(dc.astype(jnp.int32), 0, _DBINS - 2)
        whi = dc - dlo.astype(jnp.float32)
        wlo = 1.0 - whi

        samp_img = jnp.zeros((_Nc, P, _HEAD), jnp.float32)
        dscore = jnp.zeros((_Nc, _N, _M, _NR1, _NR2), jnp.float32)
        for dy in (0, 1):
            for dx in (0, 1):
                xi = x0 + dx
                yi = y0 + dy
                tv = ((xi >= 0) & (xi < W) & (yi >= 0) & (yi < H))
                xc = jnp.clip(xi, 0, W - 1)
                yc = jnp.clip(yi, 0, H - 1)
                tw = ((fx if dx else 1.0 - fx) * (fy if dy else 1.0 - fy)
                      * tv.astype(jnp.float32))
                pix = yc * W + xc
                ridx = (pix * _M + m_idx).reshape(_Nc, P)
                g = jnp.take_along_axis(featfl, ridx[:, :, None], axis=1)
                samp_img = samp_img + tw.reshape(_Nc, P)[:, :, None] * g
                didx = pix * _DBINS + dlo
                slo = jnp.take_along_axis(dmapfl, didx.reshape(_Nc, P), axis=1)
                shi = jnp.take_along_axis(dmapfl, (didx + 1).reshape(_Nc, P), axis=1)
                dscore = dscore + tw * (wlo * slo.reshape(dscore.shape)
                                        + whi * shi.reshape(dscore.shape))

        coef = (dscore * wl).reshape(_Nc, P)
        weighted = samp_img * coef[:, :, None]
        acc = acc + weighted.reshape(_Nc, _N, _M, _NR1, _NR2, _HEAD).sum(axis=(3, 4))

    x_out = acc.sum(axis=0).reshape(_N, _C)
    out = _out_projection(x_out, W_out, b_out, q)
    return out.reshape(_B, _N, _C)


# speedup vs baseline: 31.9461x; 26.7796x over previous
import functools

import jax
import jax.numpy as jnp
from jax import lax
from jax.experimental import pallas as pl
from jax.experimental.pallas import tpu as pltpu
from jax.experimental.pallas import tpu_sc as plsc

_B, _Nc, _N, _C = 1, 6, 1024, 256
_M, _HEAD, _L, _NR1, _NR2, _DBINS = 8, 32, 4, 2, 4, 32
_DMIN, _DMAX = 1.0, 60.0
_LEVEL_HW = [(32, 88), (16, 44), (8, 22), (4, 11)]

_NG = _Nc * _L
_PCL = _N * _M * _NR1 * _NR2
_DTAB = 90112
_NU = _N * _M
_PPU = _Nc * _L * _NR1 * _NR2
_RPU = _PPU * 4
_FROWS = sum(_Nc * h * w * _M for h, w in _LEVEL_HW)
_ACHUNK = 1024
_UPW = _NU // 32


def _prep(query, gaussian_means, lidar2img, W_off3d, b_off3d, W_uvd, b_uvd,
          W_attn, b_attn, H_orig, W_orig):
    q = query[0]
    gm = gaussian_means[0]
    l2i = lidar2img[0]

    off3d = (q @ W_off3d + b_off3d).reshape(_N, _NR1, 3)
    ref3d = gm[:, None, :] + off3d
    homo = jnp.concatenate(
        [ref3d, jnp.ones((_N, _NR1, 1), q.dtype)], -1).reshape(_N * _NR1, 4)
    proj = jnp.einsum('cij,nj->cni', l2i, homo)
    depth = jnp.maximum(proj[..., 2], 1e-5)
    u = proj[..., 0] / depth / W_orig
    v = proj[..., 1] / depth / H_orig
    dn = jnp.clip((depth - _DMIN) / (_DMAX - _DMIN), 0.0, 1.0)
    valid = ((u >= 0) & (u <= 1) & (v >= 0) & (v <= 1) & (depth > _DMIN))
    u = u.reshape(_Nc, _N, _NR1)
    v = v.reshape(_Nc, _N, _NR1)
    dn = dn.reshape(_Nc, _N, _NR1)
    valid = valid.reshape(_Nc, _N, _NR1)

    offs = (q @ W_uvd + b_uvd).reshape(_N, _M, _L, _NR1, _NR2, 3)
    attn = (q @ W_attn + b_attn).reshape(_N, _M, _L * _NR1 * _NR2)
    attn = jax.nn.softmax(attn, axis=-1).reshape(_N, _M, _L, _NR1, _NR2)

    su = u[:, :, None, None, :, None] + offs[None, :, :, :, :, :, 0]
    sv = v[:, :, None, None, :, None] + offs[None, :, :, :, :, :, 1]
    sd = dn[:, :, None, None, :, None] + offs[None, :, :, :, :, :, 2]
    aw = attn[None] * valid[:, :, None, None, :, None].astype(jnp.float32)

    m_idx = jnp.arange(_M, dtype=jnp.int32)[None, None, :, None, None]
    c_idx = jnp.arange(_Nc, dtype=jnp.int32)[:, None, None, None, None]

    aidx_l, aw_l, fidx_l = [], [], []
    base = 0
    for l, (H, W) in enumerate(_LEVEL_HW):
        x = su[:, :, :, l] * W - 0.5
        y = sv[:, :, :, l] * H - 0.5
        d = jnp.clip(sd[:, :, :, l], 0.0, 1.0)
        x0 = jnp.floor(x)
        y0 = jnp.floor(y)
        fx = x - x0
        fy = y - y0
        dc = d * (_DBINS - 1)
        dlo = jnp.clip(dc.astype(jnp.int32), 0, _DBINS - 2)
        whi = dc - dlo.astype(jnp.float32)
        wlo = 1.0 - whi

        didx_t, fidx_t, tw_t = [], [], []
        for dy in (0, 1):
            for dx in (0, 1):
                xi = x0 + dx
                yi = y0 + dy
                tv = ((xi >= 0) & (xi <= W - 1) & (yi >= 0) & (yi <= H - 1))
                xc = jnp.clip(xi, 0, W - 1).astype(jnp.int32)
                yc = jnp.clip(yi, 0, H - 1).astype(jnp.int32)
                pix = yc * W + xc
                didx_t.append(pix * _DBINS + dlo)
                fidx_t.append(base + ((c_idx * H + yc) * W + xc) * _M + m_idx)
                tw_t.append((fx if dx else 1.0 - fx)
                            * (fy if dy else 1.0 - fy)
                            * tv.astype(jnp.float32))
        base += _Nc * H * W * _M
        aidx_l.append(jnp.stack(didx_t, 1).reshape(_Nc, 1, 4, _PCL))
        aw_l.append(jnp.stack(tw_t + [wlo, whi, aw[:, :, :, l]], 1)
                    .reshape(_Nc, 1, 7, _PCL))
        fidx_l.append(jnp.stack(fidx_t, -1)[:, :, :, None])

    A_idx = jnp.concatenate(aidx_l, 1).reshape(_NG, 4, _PCL)
    A_w = jnp.concatenate(aw_l, 1).reshape(_NG, 7, _PCL)
    fidx = jnp.concatenate(fidx_l, 3).transpose(1, 2, 0, 3, 4, 5, 6)
    fidx = fidx.reshape(_NU, 6, 128)
    return A_idx, A_w, fidx


def _dmap_table(dmaps):
    cols = []
    for l, (H, W) in enumerate(_LEVEL_HW):
        t = dmaps[l].reshape(_Nc, _DBINS, H * W).transpose(0, 2, 1)
        t = t.reshape(_Nc, H * W * _DBINS)
        t = jnp.pad(t, ((0, 0), (0, _DTAB - H * W * _DBINS)))
        cols.append(t[:, None])
    return jnp.concatenate(cols, 1).reshape(_NG, _DTAB)


def _feat_table(feats):
    rows = []
    for l, (H, W) in enumerate(_LEVEL_HW):
        t = feats[l].reshape(_Nc, _M, _HEAD, H * W).transpose(0, 3, 1, 2)
        rows.append(t.reshape(_Nc * H * W * _M, _HEAD))
    return jnp.concatenate(rows, 0)


@functools.lru_cache(maxsize=None)
def _pass_a():
    mesh = plsc.VectorSubcoreMesh(core_axis_name="c", subcore_axis_name="s")
    ncores = 2

    @functools.partial(
        pl.kernel, mesh=mesh,
        compiler_params=pltpu.CompilerParams(needs_layout_passes=False),
        out_type=jax.ShapeDtypeStruct((_NG, 4, _PCL), jnp.float32),
        scratch_types=[
            pltpu.VMEM((_DTAB,), jnp.float32),
            pltpu.VMEM((4, _ACHUNK), jnp.int32),
            pltpu.VMEM((7, _ACHUNK), jnp.float32),
            pltpu.VMEM((4, _ACHUNK), jnp.float32),
        ],
    )
    def k(dtab_hbm, aidx_hbm, aw_hbm, wout_hbm, tab, ib_i, ib_w, ob):
        wid = lax.axis_index("s") * ncores + lax.axis_index("c")

        @pl.when(wid < _NG)
        def _():
            pltpu.sync_copy(dtab_hbm.at[wid], tab)

            def chunk(ci, _):
                o = ci * _ACHUNK
                pltpu.sync_copy(aidx_hbm.at[wid, :, pl.ds(o, _ACHUNK)], ib_i)
                pltpu.sync_copy(aw_hbm.at[wid, :, pl.ds(o, _ACHUNK)], ib_w)

                def grp(i, _):
                    sl = pl.ds(i * 16, 16)
                    wlo = ib_w[4, sl]
                    whi = ib_w[5, sl]
                    awv = ib_w[6, sl]
                    tws = [ib_w[t, sl] for t in range(4)]
                    s = jnp.zeros((16,), jnp.float32)
                    for t in range(4):
                        idx = ib_i[t, sl]
                        slo = plsc.load_gather(tab, [idx])
                        shi = plsc.load_gather(tab, [idx + 1])
                        s = s + tws[t] * (wlo * slo + whi * shi)
                    qv = awv * s
                    for t in range(4):
                        ob[t, sl] = qv * tws[t]
                    return 0

                lax.fori_loop(0, _ACHUNK // 16, grp, 0)
                pltpu.sync_copy(ob, wout_hbm.at[wid, :, pl.ds(o, _ACHUNK)])
                return 0

            lax.fori_loop(0, _PCL // _ACHUNK, chunk, 0)

    return k


@functools.lru_cache(maxsize=None)
def _pass_b():
    mesh = plsc.VectorSubcoreMesh(core_axis_name="c", subcore_axis_name="s")
    ncores = 2

    @functools.partial(
        pl.kernel, mesh=mesh,
        compiler_params=pltpu.CompilerParams(use_tc_tiling_on_sc=False),
        out_type=jax.ShapeDtypeStruct((_NU, _HEAD), jnp.float32),
        scratch_types=[
            pltpu.VMEM((6, 128), jnp.int32),
            pltpu.VMEM((_RPU, _HEAD), jnp.float32),
            pltpu.VMEM((_RPU,), jnp.float32),
            pltpu.VMEM((_UPW, _HEAD), jnp.float32),
            pltpu.SemaphoreType.DMA,
        ],
    )
    def k(ftab_hbm, fidx_hbm, wgt_hbm, out_hbm, idxb, rowsb, wb, outb, sem):
        wid = lax.axis_index("s") * ncores + lax.axis_index("c")

        def unit(u, _):
            unit_id = wid * _UPW + u
            pltpu.sync_copy(fidx_hbm.at[unit_id], idxb)
            cps = [
                pltpu.async_copy(
                    ftab_hbm.at[idxb.at[j]],
                    rowsb.at[pl.ds(j * 128, 128)], sem)
                for j in range(6)
            ]
            pltpu.sync_copy(wgt_hbm.at[unit_id], wb)
            for cp in cps:
                cp.wait()

            def pt4(p4, carry):
                a0, a1 = carry
                wv = wb[pl.ds(p4 * 16, 16)]
                r = p4 * 16
                for j in range(16):
                    wt = wv[j]
                    a0 = a0 + wt * rowsb[r + j, pl.ds(0, 16)]
                    a1 = a1 + wt * rowsb[r + j, pl.ds(16, 16)]
                return (a0, a1)

            z = jnp.zeros((16,), jnp.float32)
            a0, a1 = lax.fori_loop(0, _PPU // 4, pt4, (z, z))
            outb[u, pl.ds(0, 16)] = a0
            outb[u, pl.ds(16, 16)] = a1
            return 0

        lax.fori_loop(0, _UPW, unit, 0)
        pltpu.sync_copy(outb, out_hbm.at[pl.ds(wid * _UPW, _UPW)])

    return k


def _out_proj_body(x_ref, w_ref, b_ref, q_ref, o_ref):
    o_ref[...] = (
        jnp.dot(x_ref[...], w_ref[...], preferred_element_type=jnp.float32)
        + b_ref[...][None, :] + q_ref[...]
    )


def kernel(query, gaussian_means, lidar2img, feat0, feat1, feat2, feat3,
           dmap0, dmap1, dmap2, dmap3, W_off3d, b_off3d, W_uvd, b_uvd,
           W_attn, b_attn, W_out, b_out, H_orig, W_orig):
    A_idx, A_w, fidx = _prep(query, gaussian_means, lidar2img, W_off3d,
                             b_off3d, W_uvd, b_uvd, W_attn, b_attn,
                             H_orig, W_orig)
    dtab = _dmap_table([dmap0, dmap1, dmap2, dmap3])
    ftab = _feat_table([feat0, feat1, feat2, feat3])

    wA = _pass_a()(dtab, A_idx, A_w)
    wB = (wA.reshape(_Nc, _L, 4, _N, _M, _NR1, _NR2)
          .transpose(3, 4, 0, 1, 5, 6, 2).reshape(_NU, _RPU))
    x = _pass_b()(ftab, fidx, wB).reshape(_N, _C)

    out = pl.pallas_call(
        _out_proj_body,
        out_shape=jax.ShapeDtypeStruct((_N, _C), jnp.float32),
    )(x, W_out, b_out, query[0])
    return out.reshape(_B, _N, _C)


# speedup vs baseline: 42.8604x; 1.3416x over previous
import functools

import jax
import jax.numpy as jnp
from jax import lax
from jax.experimental import pallas as pl
from jax.experimental.pallas import tpu as pltpu
from jax.experimental.pallas import tpu_sc as plsc

_B, _Nc, _N, _C = 1, 6, 1024, 256
_M, _HEAD, _L, _NR1, _NR2, _DBINS = 8, 32, 4, 2, 4, 32
_DMIN, _DMAX = 1.0, 60.0
_LEVEL_HW = [(32, 88), (16, 44), (8, 22), (4, 11)]

_NG = _Nc * _L
_PCL = _N * _M * _NR1 * _NR2
_R = _NR1 * _NR2
_DTAB = 90112
_NU = _N * _M
_PPU = _Nc * _L * _R
_RPU = _PPU * 4
_FROWS = sum(_Nc * h * w * _M for h, w in _LEVEL_HW)
_ACHUNK = 1024
_AU = _ACHUNK // _R
_UPW = _NU // 32


def _prep(query, gaussian_means, lidar2img, W_off3d, b_off3d, W_uvd, b_uvd,
          W_attn, b_attn, H_orig, W_orig):
    q = query[0]
    gm = gaussian_means[0]
    l2i = lidar2img[0]

    off3d = (q @ W_off3d + b_off3d).reshape(_N, _NR1, 3)
    ref3d = gm[:, None, :] + off3d
    homo = jnp.concatenate(
        [ref3d, jnp.ones((_N, _NR1, 1), q.dtype)], -1).reshape(_N * _NR1, 4)
    proj = jnp.einsum('cij,nj->cni', l2i, homo)
    depth = jnp.maximum(proj[..., 2], 1e-5)
    u = proj[..., 0] / depth / W_orig
    v = proj[..., 1] / depth / H_orig
    dn = jnp.clip((depth - _DMIN) / (_DMAX - _DMIN), 0.0, 1.0)
    valid = ((u >= 0) & (u <= 1) & (v >= 0) & (v <= 1) & (depth > _DMIN))
    u = u.reshape(_Nc, _N, _NR1)
    v = v.reshape(_Nc, _N, _NR1)
    dn = dn.reshape(_Nc, _N, _NR1)
    valid = valid.reshape(_Nc, _N, _NR1)

    offs = (q @ W_uvd + b_uvd).reshape(_N, _M, _L, _NR1, _NR2, 3)
    attn = (q @ W_attn + b_attn).reshape(_N, _M, _L * _NR1 * _NR2)
    attn = jax.nn.softmax(attn, axis=-1).reshape(_N, _M, _L, _NR1, _NR2)

    su = u[:, :, None, None, :, None] + offs[None, :, :, :, :, :, 0]
    sv = v[:, :, None, None, :, None] + offs[None, :, :, :, :, :, 1]
    sd = dn[:, :, None, None, :, None] + offs[None, :, :, :, :, :, 2]
    aw = attn[None] * valid[:, :, None, None, :, None].astype(jnp.float32)

    m_idx = jnp.arange(_M, dtype=jnp.int32)[None, None, :, None, None]
    c_idx = jnp.arange(_Nc, dtype=jnp.int32)[:, None, None, None, None]

    aifx_l, aw_l = [], []
    base = 0
    for l, (H, W) in enumerate(_LEVEL_HW):
        x = su[:, :, :, l] * W - 0.5
        y = sv[:, :, :, l] * H - 0.5
        d = jnp.clip(sd[:, :, :, l], 0.0, 1.0)
        x0 = jnp.floor(x)
        y0 = jnp.floor(y)
        fx = x - x0
        fy = y - y0
        dc = d * (_DBINS - 1)
        dlo = jnp.clip(dc.astype(jnp.int32), 0, _DBINS - 2)
        whi = dc - dlo.astype(jnp.float32)
        wlo = 1.0 - whi

        didx_t, fidx_t, tw_t = [], [], []
        for dy in (0, 1):
            for dx in (0, 1):
                xi = x0 + dx
                yi = y0 + dy
                tv = ((xi >= 0) & (xi <= W - 1) & (yi >= 0) & (yi <= H - 1))
                xc = jnp.clip(xi, 0, W - 1).astype(jnp.int32)
                yc = jnp.clip(yi, 0, H - 1).astype(jnp.int32)
                pix = yc * W + xc
                didx_t.append(pix * _DBINS + dlo)
                fidx_t.append(base + ((c_idx * H + yc) * W + xc) * _M + m_idx)
                tw_t.append((fx if dx else 1.0 - fx)
                            * (fy if dy else 1.0 - fy)
                            * tv.astype(jnp.float32))
        base += _Nc * H * W * _M
        aifx_l.append(jnp.stack(didx_t + fidx_t, 1).reshape(_Nc, 1, 8, _PCL))
        aw_l.append(jnp.stack(tw_t + [wlo, whi, aw[:, :, :, l]], 1)
                    .reshape(_Nc, 1, 7, _PCL))

    A_ifx = jnp.concatenate(aifx_l, 1).reshape(_NG, 8, _PCL)
    A_w = jnp.concatenate(aw_l, 1).reshape(_NG, 7, _PCL)
    return A_ifx, A_w


def _dmap_table(dmaps):
    cols = []
    for l, (H, W) in enumerate(_LEVEL_HW):
        t = dmaps[l].reshape(_Nc, _DBINS, H * W).transpose(0, 2, 1)
        t = t.reshape(_Nc, H * W * _DBINS)
        t = jnp.pad(t, ((0, 0), (0, _DTAB - H * W * _DBINS)))
        cols.append(t[:, None])
    return jnp.concatenate(cols, 1).reshape(_NG, _DTAB)


def _feat_table(feats):
    rows = []
    for l, (H, W) in enumerate(_LEVEL_HW):
        t = feats[l].reshape(_Nc, _M, _HEAD, H * W).transpose(0, 3, 1, 2)
        rows.append(t.reshape(_Nc * H * W * _M, _HEAD))
    return jnp.concatenate(rows, 0)


@functools.lru_cache(maxsize=None)
def _pass_a():
    mesh = plsc.VectorSubcoreMesh(core_axis_name="c", subcore_axis_name="s")
    ncores = 2

    @functools.partial(
        pl.kernel, mesh=mesh,
        compiler_params=pltpu.CompilerParams(needs_layout_passes=False,
                                             use_tc_tiling_on_sc=False),
        out_type=(jax.ShapeDtypeStruct((_NU, _RPU), jnp.float32),
                  jax.ShapeDtypeStruct((_NU, _RPU), jnp.int32)),
        scratch_types=[
            pltpu.VMEM((_DTAB,), jnp.float32),
            pltpu.VMEM((8, _ACHUNK), jnp.int32),
            pltpu.VMEM((7, _ACHUNK), jnp.float32),
            pltpu.VMEM((_AU, 32), jnp.float32),
            pltpu.VMEM((_AU, 32), jnp.int32),
        ],
    )
    def k(dtab_hbm, aifx_hbm, aw_hbm, wgt_hbm, fout_hbm, tab, ib_i, ib_w,
          ob_w, ob_f):
        wid = lax.axis_index("s") * ncores + lax.axis_index("c")

        @pl.when(wid < _NG)
        def _():
            pltpu.sync_copy(dtab_hbm.at[wid], tab)
            lanes = lax.iota(jnp.int32, 16)
            rowp = lax.shift_right_logical(lanes, 3)
            colp = (lanes & 7) * 4

            def chunk(ci, _):
                o = ci * _ACHUNK
                pltpu.sync_copy(aifx_hbm.at[wid, :, pl.ds(o, _ACHUNK)], ib_i)
                pltpu.sync_copy(aw_hbm.at[wid, :, pl.ds(o, _ACHUNK)], ib_w)

                def grp(i, _):
                    sl = pl.ds(i * 16, 16)
                    wlo = ib_w[4, sl]
                    whi = ib_w[5, sl]
                    awv = ib_w[6, sl]
                    tws = [ib_w[t, sl] for t in range(4)]
                    s = jnp.zeros((16,), jnp.float32)
                    for t in range(4):
                        idx = ib_i[t, sl]
                        slo = plsc.load_gather(tab, [idx])
                        shi = plsc.load_gather(tab, [idx + 1])
                        s = s + tws[t] * (wlo * slo + whi * shi)
                    qv = awv * s
                    rowv = i * 2 + rowp
                    for t in range(4):
                        plsc.store_scatter(ob_w, [rowv, colp + t], qv * tws[t])
                        plsc.store_scatter(ob_f, [rowv, colp + t],
                                           ib_i[4 + t, sl])
                    return 0

                lax.fori_loop(0, _ACHUNK // 16, grp, 0)
                ub = ci * _AU
                pltpu.sync_copy(
                    ob_w, wgt_hbm.at[pl.ds(ub, _AU), pl.ds(wid * 32, 32)])
                pltpu.sync_copy(
                    ob_f, fout_hbm.at[pl.ds(ub, _AU), pl.ds(wid * 32, 32)])
                return 0

            lax.fori_loop(0, _PCL // _ACHUNK, chunk, 0)

    return k


@functools.lru_cache(maxsize=None)
def _pass_b():
    mesh = plsc.VectorSubcoreMesh(core_axis_name="c", subcore_axis_name="s")
    ncores = 2

    @functools.partial(
        pl.kernel, mesh=mesh,
        compiler_params=pltpu.CompilerParams(use_tc_tiling_on_sc=False),
        out_type=jax.ShapeDtypeStruct((_NU, _HEAD), jnp.float32),
        scratch_types=[
            pltpu.VMEM((2, 6, 128), jnp.int32),
            pltpu.VMEM((2, _RPU, _HEAD), jnp.float32),
            pltpu.VMEM((2, _RPU), jnp.float32),
            pltpu.VMEM((_UPW, _HEAD), jnp.float32),
            pltpu.SemaphoreType.DMA,
            pltpu.SemaphoreType.DMA,
        ],
    )
    def k(ftab_hbm, fidx_hbm, wgt_hbm, out_hbm, idxb, rowsb, wb, outb,
          sem0, sem1):
        wid = lax.axis_index("s") * ncores + lax.axis_index("c")
        sems = (sem0, sem1)

        def fetch(unit_id, buf, sem):
            pltpu.sync_copy(fidx_hbm.at[unit_id], idxb.at[buf])
            for j in range(6):
                pltpu.async_copy(
                    ftab_hbm.at[idxb.at[buf, j]],
                    rowsb.at[buf, pl.ds(j * 128, 128)], sem)
            pltpu.sync_copy(wgt_hbm.at[unit_id], wb.at[buf])

        def compute(u_local, buf, sem):
            for j in range(6):
                pltpu.make_async_copy(
                    ftab_hbm.at[idxb.at[buf, j]],
                    rowsb.at[buf, pl.ds(j * 128, 128)], sem).wait()

            def pt4(p4, carry):
                a0, a1 = carry
                wv = wb[buf, pl.ds(p4 * 16, 16)]
                r = p4 * 16
                for j in range(16):
                    wt = wv[j]
                    a0 = a0 + wt * rowsb[buf, r + j, pl.ds(0, 16)]
                    a1 = a1 + wt * rowsb[buf, r + j, pl.ds(16, 16)]
                return (a0, a1)

            z = jnp.zeros((16,), jnp.float32)
            a0, a1 = lax.fori_loop(0, _PPU // 4, pt4, (z, z))
            outb[u_local, pl.ds(0, 16)] = a0
            outb[u_local, pl.ds(16, 16)] = a1

        base = wid * _UPW
        fetch(base, 0, sem0)

        def pair(i, _):
            u0 = i * 2
            fetch(base + u0 + 1, 1, sem1)
            compute(u0, 0, sem0)

            @pl.when(i < _UPW // 2 - 1)
            def _():
                fetch(base + u0 + 2, 0, sem0)

            compute(u0 + 1, 1, sem1)
            return 0

        lax.fori_loop(0, _UPW // 2, pair, 0)
        pltpu.sync_copy(outb, out_hbm.at[pl.ds(base, _UPW)])

    return k


def _out_proj_body(x_ref, w_ref, b_ref, q_ref, o_ref):
    o_ref[...] = (
        jnp.dot(x_ref[...], w_ref[...], preferred_element_type=jnp.float32)
        + b_ref[...][None, :] + q_ref[...]
    )


def kernel(query, gaussian_means, lidar2img, feat0, feat1, feat2, feat3,
           dmap0, dmap1, dmap2, dmap3, W_off3d, b_off3d, W_uvd, b_uvd,
           W_attn, b_attn, W_out, b_out, H_orig, W_orig):
    A_ifx, A_w = _prep(query, gaussian_means, lidar2img, W_off3d,
                       b_off3d, W_uvd, b_uvd, W_attn, b_attn,
                       H_orig, W_orig)
    dtab = _dmap_table([dmap0, dmap1, dmap2, dmap3])
    ftab = _feat_table([feat0, feat1, feat2, feat3])

    wgt, fidx = _pass_a()(dtab, A_ifx, A_w)
    x = _pass_b()(ftab, fidx.reshape(_NU, 6, 128), wgt).reshape(_N, _C)

    out = pl.pallas_call(
        _out_proj_body,
        out_shape=jax.ShapeDtypeStruct((_N, _C), jnp.float32),
    )(x, W_out, b_out, query[0])
    return out.reshape(_B, _N, _C)
